# baseline (speedup 1.0000x reference)
import jax
import jax.numpy as jnp
from jax import lax
from jax.experimental import pallas as pl
from jax.experimental.pallas import tpu as pltpu
from jax.experimental.pallas import tpu_sc as plsc

T = 26
B = 4096
ROWS = 100000
D = 64
L = 20

NC, NS = 2, 16
NW = NC * NS
NP = T // 2
CHUNK_BAGS = 16
HALF_ROWS = CHUNK_BAGS * L
CHUNK_ROWS = 2 * HALF_ROWS
CHUNKS_PER_PAIR = B // CHUNK_BAGS
N_CHUNKS = NP * CHUNKS_PER_PAIR // NW
GATHER_BLK = 128
N_GATHERS = CHUNK_ROWS // GATHER_BLK

PACK_COLS = 2048
NB = 635
S_SPLIT = NB * PACK_COLS


def _pack_body(lo_ref, hi_ref, out_ref):
    out_ref[...] = jnp.concatenate([lo_ref[...].T, hi_ref[...].T], axis=1)


def _body(idx_hbm, w_hbm, out_hbm, idx_v, rows_v, out_v, sem_g0, sem_g1,
          sem_o0, sem_o1):
    sem_g = (sem_g0, sem_g1)
    sem_o = (sem_o0, sem_o1)
    wid = lax.axis_index("s") * NC + lax.axis_index("c")

    def coords(c):
        g = wid * N_CHUNKS + c
        p = g // CHUNKS_PER_PAIR
        b0 = (g - p * CHUNKS_PER_PAIR) * CHUNK_BAGS
        return p, b0

    def gather_descs(buf):
        return [
            pltpu.make_async_copy(
                w_hbm.at[idx_v.at[buf, pl.ds(j * GATHER_BLK, GATHER_BLK)]],
                rows_v.at[buf, pl.ds(j * GATHER_BLK, GATHER_BLK), :],
                sem_g[buf],
            )
            for j in range(N_GATHERS)
        ]

    def out_desc(c, buf):
        p, b0 = coords(c)
        return pltpu.make_async_copy(
            out_v.at[buf],
            out_hbm.at[pl.ds(b0, CHUNK_BAGS), pl.ds(p * 2 * D, 2 * D)],
            sem_o[buf],
        )

    def stage(c, buf):
        p, b0 = coords(c)
        t0 = 2 * p
        pltpu.sync_copy(
            idx_hbm.at[pl.ds((t0 * B + b0) * L, HALF_ROWS)],
            idx_v.at[buf, pl.ds(0, HALF_ROWS)],
        )
        pltpu.sync_copy(
            idx_hbm.at[pl.ds(((t0 + 1) * B + b0) * L, HALF_ROWS)],
            idx_v.at[buf, pl.ds(HALF_ROWS, HALF_ROWS)],
        )
        for h in range(2):
            tbase = (t0 + h) * ROWS
            for k in range(HALF_ROWS // 16):
                sl = pl.ds(h * HALF_ROWS + k * 16, 16)
                gidx = idx_v[buf, sl] + tbase
                idx_v[buf, sl] = 2 * gidx - jnp.where(
                    gidx >= S_SPLIT, 2 * S_SPLIT - 1, 0)
        for cp in gather_descs(buf):
            cp.start()

    def accum(c, buf):
        def bag_body(b, _):
            for h in range(2):
                r0 = h * HALF_ROWS + b * L
                acc = [rows_v[buf, r0, pl.ds(k * 16, 16)] for k in range(D // 16)]
                for l in range(1, L):
                    for k in range(D // 16):
                        acc[k] = acc[k] + rows_v[buf, r0 + l, pl.ds(k * 16, 16)]
                for k in range(D // 16):
                    out_v[buf, b, pl.ds(h * D + k * 16, 16)] = acc[k]
            return _

        lax.fori_loop(0, CHUNK_BAGS, bag_body, 0)
        out_desc(c, buf).start()

    stage(0, 0)

    def pair_body(pp, _):
        for par in range(2):
            c = pp * 2 + par
            buf = par
            nxt = c + 1

            @pl.when(nxt < N_CHUNKS)
            def _stage_next():
                stage(nxt, 1 - buf)

            for cp in gather_descs(buf):
                cp.wait()


            @pl.when(c >= 2)
            def _drain_out():
                out_desc(c, buf).wait()

            accum(c, buf)
        return _

    lax.fori_loop(0, N_CHUNKS // 2, pair_body, 0)
    for buf in range(2):
        out_desc(N_CHUNKS - 2 + buf, buf).wait()


@jax.jit
def kernel(indices, offsets, weights):
    del offsets
    wt = weights.T
    wp = pl.pallas_call(
        _pack_body,
        grid=(NB,),
        in_specs=[
            pl.BlockSpec((D, PACK_COLS), lambda i: (0, i)),
            pl.BlockSpec((D, PACK_COLS), lambda i: (0, NB + i)),
        ],
        out_specs=pl.BlockSpec((PACK_COLS, 2 * D), lambda i: (i, 0)),
        out_shape=jax.ShapeDtypeStruct((S_SPLIT, 2 * D), jnp.float32),
    )(wt, wt)
    run = pl.kernel(
        _body,
        out_type=jax.ShapeDtypeStruct((B, T * D), jnp.float32),
        mesh=plsc.VectorSubcoreMesh(core_axis_name="c", subcore_axis_name="s"),
        scratch_types=[
            pltpu.VMEM((2, CHUNK_ROWS), jnp.int32),
            pltpu.VMEM((2, CHUNK_ROWS, D), jnp.float32),
            pltpu.VMEM((2, CHUNK_BAGS, 2 * D), jnp.float32),
            pltpu.SemaphoreType.DMA,
            pltpu.SemaphoreType.DMA,
            pltpu.SemaphoreType.DMA,
            pltpu.SemaphoreType.DMA,
        ],
        compiler_params=pltpu.CompilerParams(use_tc_tiling_on_sc=False),
    )
    return run(indices, wp.reshape(2 * S_SPLIT, D))
